# baseline (speedup 1.0000x reference)
import functools

import jax
import jax.numpy as jnp
from jax import lax
from jax.experimental import pallas as pl
from jax.experimental.pallas import tpu as pltpu

D_MODEL = 1024
N_GROUP = 64
K_C = 8
K_QK = 4
K_V = 6
S_CHUNK = 256


def _sparsify(rows, k):
    sparse = jnp.zeros_like(rows)
    work = rows
    n = rows.shape[-1]
    iota = lax.broadcasted_iota(jnp.int32, rows.shape, rows.ndim - 1)
    for _ in range(k):
        m = jnp.max(work, axis=-1, keepdims=True)
        eq = work == m
        min_idx = jnp.min(jnp.where(eq, iota, n), axis=-1, keepdims=True)
        first = iota == min_idx
        sparse = jnp.where(first, work, sparse)
        work = jnp.where(first, -jnp.inf, work)
    total = jnp.sum(sparse, axis=-1, keepdims=True)
    return sparse / (total + 1e-8)


def _body(x_ref, imp_ref, w_ref, bp_ref, emb_ref,
          cw_ref, qw_ref, kw_ref, vw_ref,
          cacc, qkacc, vacc, *, nchunk, nb):
    c = pl.program_id(0)
    t = nb * S_CHUNK

    xf = x_ref[...].reshape(t, D_MODEL)
    ht = lax.dot_general(w_ref[...], xf, (((1,), (1,)), ((), ())),
                         preferred_element_type=jnp.float32)
    ht = ht + bp_ref[...]

    emb = emb_ref[...]
    norm = jnp.sqrt(jnp.sum(emb * emb, axis=-1, keepdims=True))
    emb_n = emb / jnp.maximum(norm, 1e-12)

    lt = lax.dot_general(emb_n, ht, (((1,), (0,)), ((), ())),
                         preferred_element_type=jnp.float32)

    imps = [imp_ref[b, 0, pl.ds(c * S_CHUNK, S_CHUNK)][None, :]
            for b in range(nb)]

    def group_contrib(lg):
        m = jnp.max(lg, axis=0, keepdims=True)
        e = jnp.exp(lg - m)
        s = jnp.sum(e, axis=0, keepdims=True)
        p = e / s
        cons = [lax.dot_general(imps[b], p[:, b * S_CHUNK:(b + 1) * S_CHUNK],
                                (((1,), (1,)), ((), ())),
                                preferred_element_type=jnp.float32)
                for b in range(nb)]
        return jnp.concatenate(cons, axis=0)

    con_c = group_contrib(lt[0:64])
    con_qk = group_contrib(lt[64:128])
    con_v = group_contrib(lt[128:192])

    @pl.when(c == 0)
    def _init():
        cacc[...] = con_c
        qkacc[...] = con_qk
        vacc[...] = con_v

    @pl.when(c != 0)
    def _accum():
        cacc[...] += con_c
        qkacc[...] += con_qk
        vacc[...] += con_v

    @pl.when(c == nchunk - 1)
    def _finish():
        cw_ref[...] = _sparsify(cacc[...], K_C)
        qk = _sparsify(qkacc[...], K_QK)
        qw_ref[...] = qk
        kw_ref[...] = qk
        vw_ref[...] = _sparsify(vacc[...], K_V)


@jax.jit
def kernel(x, importance, W_proj, b_proj, neuron_emb):
    B, S, _ = x.shape
    nchunk = S // S_CHUNK
    bp = b_proj[:, None]
    imp3 = importance[:, None, :]

    out_shape = [jax.ShapeDtypeStruct((B, N_GROUP), jnp.float32)] * 4
    out_spec = pl.BlockSpec((B, N_GROUP), lambda c: (0, 0))

    outs = pl.pallas_call(
        functools.partial(_body, nchunk=nchunk, nb=B),
        grid=(nchunk,),
        in_specs=[
            pl.BlockSpec((B, S_CHUNK, D_MODEL), lambda c: (0, c, 0)),
            pl.BlockSpec((B, 1, S), lambda c: (0, 0, 0)),
            pl.BlockSpec(W_proj.shape, lambda c: (0, 0)),
            pl.BlockSpec(bp.shape, lambda c: (0, 0)),
            pl.BlockSpec(neuron_emb.shape, lambda c: (0, 0)),
        ],
        out_specs=[out_spec] * 4,
        out_shape=out_shape,
        scratch_shapes=[pltpu.VMEM((B, N_GROUP), jnp.float32)] * 3,
    )(x, imp3, W_proj, bp, neuron_emb)
    return tuple(outs)


# speedup vs baseline: 1.0448x; 1.0448x over previous
import functools

import jax
import jax.numpy as jnp
from jax import lax
from jax.experimental import pallas as pl
from jax.experimental.pallas import tpu as pltpu

D_MODEL = 1024
N_GROUP = 64
K_C = 8
K_QK = 4
K_V = 6
S_CHUNK = 1024


def _sparsify(rows, k):
    sparse = jnp.zeros_like(rows)
    work = rows
    n = rows.shape[-1]
    iota = lax.broadcasted_iota(jnp.int32, rows.shape, rows.ndim - 1)
    for _ in range(k):
        m = jnp.max(work, axis=-1, keepdims=True)
        eq = work == m
        min_idx = jnp.min(jnp.where(eq, iota, n), axis=-1, keepdims=True)
        first = iota == min_idx
        sparse = jnp.where(first, work, sparse)
        work = jnp.where(first, -jnp.inf, work)
    total = jnp.sum(sparse, axis=-1, keepdims=True)
    return sparse / (total + 1e-8)


def _body(x_ref, imp_ref, w_ref, bp_ref, emb_ref,
          cw_ref, qw_ref, kw_ref, vw_ref,
          cacc, qkacc, vacc, *, nchunk, nb):
    c = pl.program_id(0)
    t = nb * S_CHUNK

    xf = x_ref[...].reshape(t, D_MODEL)
    ht = lax.dot_general(w_ref[...], xf, (((1,), (1,)), ((), ())),
                         preferred_element_type=jnp.float32)
    ht = ht + bp_ref[...]

    emb = emb_ref[...]
    norm = jnp.sqrt(jnp.sum(emb * emb, axis=-1, keepdims=True))
    emb_n = emb / jnp.maximum(norm, 1e-12)

    lt = lax.dot_general(emb_n, ht, (((1,), (0,)), ((), ())),
                         preferred_element_type=jnp.float32)

    imps = [imp_ref[b, 0, pl.ds(c * S_CHUNK, S_CHUNK)][None, :]
            for b in range(nb)]

    def group_contrib(lg):
        m = jnp.max(lg, axis=0, keepdims=True)
        e = jnp.exp(lg - m)
        s = jnp.sum(e, axis=0, keepdims=True)
        p = e / s
        cons = [lax.dot_general(imps[b], p[:, b * S_CHUNK:(b + 1) * S_CHUNK],
                                (((1,), (1,)), ((), ())),
                                preferred_element_type=jnp.float32)
                for b in range(nb)]
        return jnp.concatenate(cons, axis=0)

    con_c = group_contrib(lt[0:64])
    con_qk = group_contrib(lt[64:128])
    con_v = group_contrib(lt[128:192])

    @pl.when(c == 0)
    def _init():
        cacc[...] = con_c
        qkacc[...] = con_qk
        vacc[...] = con_v

    @pl.when(c != 0)
    def _accum():
        cacc[...] += con_c
        qkacc[...] += con_qk
        vacc[...] += con_v

    @pl.when(c == nchunk - 1)
    def _finish():
        cw_ref[...] = _sparsify(cacc[...], K_C)
        qk = _sparsify(qkacc[...], K_QK)
        qw_ref[...] = qk
        kw_ref[...] = qk
        vw_ref[...] = _sparsify(vacc[...], K_V)


@jax.jit
def kernel(x, importance, W_proj, b_proj, neuron_emb):
    B, S, _ = x.shape
    nchunk = S // S_CHUNK
    bp = b_proj[:, None]
    imp3 = importance[:, None, :]

    out_shape = [jax.ShapeDtypeStruct((B, N_GROUP), jnp.float32)] * 4
    out_spec = pl.BlockSpec((B, N_GROUP), lambda c: (0, 0))

    outs = pl.pallas_call(
        functools.partial(_body, nchunk=nchunk, nb=B),
        grid=(nchunk,),
        in_specs=[
            pl.BlockSpec((B, S_CHUNK, D_MODEL), lambda c: (0, c, 0)),
            pl.BlockSpec((B, 1, S), lambda c: (0, 0, 0)),
            pl.BlockSpec(W_proj.shape, lambda c: (0, 0)),
            pl.BlockSpec(bp.shape, lambda c: (0, 0)),
            pl.BlockSpec(neuron_emb.shape, lambda c: (0, 0)),
        ],
        out_specs=[out_spec] * 4,
        out_shape=out_shape,
        scratch_shapes=[pltpu.VMEM((B, N_GROUP), jnp.float32)] * 3,
    )(x, imp3, W_proj, bp, neuron_emb)
    return tuple(outs)


# speedup vs baseline: 1.0814x; 1.0351x over previous
import functools

import jax
import jax.numpy as jnp
from jax import lax
from jax.experimental import pallas as pl
from jax.experimental.pallas import tpu as pltpu

D_MODEL = 1024
N_GROUP = 64
K_C = 8
K_QK = 4
K_V = 6
S_CHUNK = 512


def _sparsify_stacked(rows, ks):
    sparse = jnp.zeros_like(rows)
    work = rows
    n = rows.shape[-1]
    iota = lax.broadcasted_iota(jnp.int32, rows.shape, rows.ndim - 1)
    row = lax.broadcasted_iota(jnp.int32, (rows.shape[0], 1), 0)
    for i in range(max(ks)):
        m = jnp.max(work, axis=-1, keepdims=True)
        eq = work == m
        min_idx = jnp.min(jnp.where(eq, iota, n), axis=-1, keepdims=True)
        first = iota == min_idx
        if any(i >= k for k in ks):
            nrow = rows.shape[0] // len(ks)
            act = jnp.zeros_like(row, dtype=jnp.bool_)
            for g, k in enumerate(ks):
                if i < k:
                    act = act | ((row >= g * nrow) & (row < (g + 1) * nrow))
            first = first & act
        sparse = jnp.where(first, work, sparse)
        work = jnp.where(first, -jnp.inf, work)
    total = jnp.sum(sparse, axis=-1, keepdims=True)
    return sparse / (total + 1e-8)


def _body(x_ref, imp_ref, w_ref, bp_ref, emb_ref,
          cw_ref, qw_ref, kw_ref, vw_ref,
          cacc, qkacc, vacc, *, nchunk, nb):
    c = pl.program_id(0)
    t = nb * S_CHUNK

    xf = x_ref[...].reshape(t, D_MODEL)
    ht = lax.dot_general(w_ref[...], xf, (((1,), (1,)), ((), ())),
                         preferred_element_type=jnp.float32)
    ht = ht + bp_ref[...]

    emb = emb_ref[...]
    norm = jnp.sqrt(jnp.sum(emb * emb, axis=-1, keepdims=True))
    emb_n = emb / jnp.maximum(norm, 1e-12)

    lt = lax.dot_general(emb_n, ht, (((1,), (0,)), ((), ())),
                         preferred_element_type=jnp.float32)

    imps = [imp_ref[b, 0, pl.ds(c * S_CHUNK, S_CHUNK)][None, :]
            for b in range(nb)]

    def group_contrib(lg):
        m = jnp.max(lg, axis=0, keepdims=True)
        e = jnp.exp(lg - m)
        s = jnp.sum(e, axis=0, keepdims=True)
        p = e / s
        cons = [lax.dot_general(imps[b], p[:, b * S_CHUNK:(b + 1) * S_CHUNK],
                                (((1,), (1,)), ((), ())),
                                preferred_element_type=jnp.float32)
                for b in range(nb)]
        return jnp.concatenate(cons, axis=0)

    con_c = group_contrib(lt[0:64])
    con_qk = group_contrib(lt[64:128])
    con_v = group_contrib(lt[128:192])

    @pl.when(c == 0)
    def _init():
        cacc[...] = con_c
        qkacc[...] = con_qk
        vacc[...] = con_v

    @pl.when(c != 0)
    def _accum():
        cacc[...] += con_c
        qkacc[...] += con_qk
        vacc[...] += con_v

    @pl.when(c == nchunk - 1)
    def _finish():
        stacked = jnp.concatenate([cacc[...], qkacc[...], vacc[...]], axis=0)
        res = _sparsify_stacked(stacked, (K_C, K_QK, K_V))
        cw_ref[...] = res[0:nb]
        qk = res[nb:2 * nb]
        qw_ref[...] = qk
        kw_ref[...] = qk
        vw_ref[...] = res[2 * nb:3 * nb]


@jax.jit
def kernel(x, importance, W_proj, b_proj, neuron_emb):
    B, S, _ = x.shape
    nchunk = S // S_CHUNK
    bp = b_proj[:, None]
    imp3 = importance[:, None, :]

    out_shape = [jax.ShapeDtypeStruct((B, N_GROUP), jnp.float32)] * 4
    out_spec = pl.BlockSpec((B, N_GROUP), lambda c: (0, 0))

    outs = pl.pallas_call(
        functools.partial(_body, nchunk=nchunk, nb=B),
        grid=(nchunk,),
        in_specs=[
            pl.BlockSpec((B, S_CHUNK, D_MODEL), lambda c: (0, c, 0)),
            pl.BlockSpec((B, 1, S), lambda c: (0, 0, 0)),
            pl.BlockSpec(W_proj.shape, lambda c: (0, 0)),
            pl.BlockSpec(bp.shape, lambda c: (0, 0)),
            pl.BlockSpec(neuron_emb.shape, lambda c: (0, 0)),
        ],
        out_specs=[out_spec] * 4,
        out_shape=out_shape,
        scratch_shapes=[pltpu.VMEM((B, N_GROUP), jnp.float32)] * 3,
    )(x, imp3, W_proj, bp, neuron_emb)
    return tuple(outs)
